# baseline (speedup 1.0000x reference)
import functools

import jax
import jax.numpy as jnp
from jax import lax
from jax.experimental import pallas as pl
from jax.experimental.pallas import tpu as pltpu
from jax.experimental.pallas import tpu_sc as plsc

NC = 2
NS = 16
NW = NC * NS
L = 16
SUB = 128
TW = 16


@functools.lru_cache(maxsize=None)
def _make(N, E, B, C, interpret=False):
    del B
    G = C // L
    NSUB = C // SUB
    T = E // C

    mesh = plsc.VectorSubcoreMesh(core_axis_name="c", subcore_axis_name="s",
                                  num_cores=NC, num_subcores=NS)

    def body(table_hbm, cs0_hbm, cs1_hbm, cs2_hbm, src_hbm, dst_hbm,
             vx_hbm, vy_hbm, vz_hbm, len_hbm,
             sidx, didx, cs0, cs1, cs2, srow, drow, vxb, vyb, vzb, lnb,
             lsem, gsem, osem):
        wid = lax.axis_index("s") * NC + lax.axis_index("c")
        n = (T - wid + NW - 1) // NW

        def issue_lin(t, p):
            chunk = wid + t * NW
            base = chunk * C
            brow = chunk * NSUB
            pltpu.async_copy(src_hbm.at[pl.ds(brow, NSUB), :], sidx[p], lsem[p])
            pltpu.async_copy(dst_hbm.at[pl.ds(brow, NSUB), :], didx[p], lsem[p])
            pltpu.async_copy(cs0_hbm.at[pl.ds(base, C)], cs0[p], lsem[p])
            pltpu.async_copy(cs1_hbm.at[pl.ds(base, C)], cs1[p], lsem[p])
            pltpu.async_copy(cs2_hbm.at[pl.ds(base, C)], cs2[p], lsem[p])

        def wait_lin(p):
            pltpu.make_async_copy(src_hbm.at[pl.ds(0, NSUB), :], sidx[p], lsem[p]).wait()
            pltpu.make_async_copy(dst_hbm.at[pl.ds(0, NSUB), :], didx[p], lsem[p]).wait()
            pltpu.make_async_copy(cs0_hbm.at[pl.ds(0, C)], cs0[p], lsem[p]).wait()
            pltpu.make_async_copy(cs1_hbm.at[pl.ds(0, C)], cs1[p], lsem[p]).wait()
            pltpu.make_async_copy(cs2_hbm.at[pl.ds(0, C)], cs2[p], lsem[p]).wait()

        def issue_gather(p):
            for j in range(NSUB):
                pltpu.async_copy(table_hbm.at[sidx[p].at[j]], srow[p].at[j], gsem[p])
                pltpu.async_copy(table_hbm.at[didx[p].at[j]], drow[p].at[j], gsem[p])

        def wait_gather(p):
            for j in range(NSUB):
                pltpu.make_async_copy(table_hbm.at[sidx[p].at[j]], srow[p].at[j], gsem[p]).wait()
                pltpu.make_async_copy(table_hbm.at[didx[p].at[j]], drow[p].at[j], gsem[p]).wait()

        def issue_out(t, p):
            base = (wid + t * NW) * C
            pltpu.async_copy(vxb[p], vx_hbm.at[pl.ds(base, C)], osem[p])
            pltpu.async_copy(vyb[p], vy_hbm.at[pl.ds(base, C)], osem[p])
            pltpu.async_copy(vzb[p], vz_hbm.at[pl.ds(base, C)], osem[p])
            pltpu.async_copy(lnb[p], len_hbm.at[pl.ds(base, C)], osem[p])

        def wait_out(p):
            pltpu.make_async_copy(vxb[p], vx_hbm.at[pl.ds(0, C)], osem[p]).wait()
            pltpu.make_async_copy(vyb[p], vy_hbm.at[pl.ds(0, C)], osem[p]).wait()
            pltpu.make_async_copy(vzb[p], vz_hbm.at[pl.ds(0, C)], osem[p]).wait()
            pltpu.make_async_copy(lnb[p], len_hbm.at[pl.ds(0, C)], osem[p]).wait()

        def compute(p):
            def group(g, carry2):
                sl = pl.ds(g * L, L)
                rows = g * L + lax.iota(jnp.int32, L)
                jv = rows >> 7
                rv = rows & (SUB - 1)

                def scol(c):
                    return plsc.load_gather(
                        srow[p], [jv, rv, jnp.full((L,), c, jnp.int32)])

                def dcol(c):
                    return plsc.load_gather(
                        drow[p], [jv, rv, jnp.full((L,), c, jnp.int32)])

                dx = dcol(0) - scol(0)
                dy = dcol(1) - scol(1)
                dz = dcol(2) - scol(2)
                c0 = cs0[p][sl]
                c1 = cs1[p][sl]
                c2 = cs2[p][sl]
                vx = dx + c0 * scol(3) + c1 * scol(6) + c2 * scol(9)
                vy = dy + c0 * scol(4) + c1 * scol(7) + c2 * scol(10)
                vz = dz + c0 * scol(5) + c1 * scol(8) + c2 * scol(11)
                s = vx * vx + vy * vy + vz * vz
                i = plsc.bitcast(s, jnp.int32)
                y = plsc.bitcast(jnp.int32(0x5F3759DF) - (i >> 1), jnp.float32)
                for _ in range(3):
                    y = y * (1.5 - 0.5 * s * y * y)
                vxb[p][sl] = vx
                vyb[p][sl] = vy
                vzb[p][sl] = vz
                lnb[p][sl] = s * y
                return carry2

            lax.fori_loop(0, G, group, 0, unroll=2)

        issue_lin(0, 0)

        @pl.when(n > 1)
        def _():
            issue_lin(1, 1)

        wait_lin(0)
        issue_gather(0)

        def step(u, carry):
            t0 = 2 * u
            t1 = t0 + 1
            t2 = t0 + 2
            t3 = t0 + 3

            wait_gather(0)

            @pl.when(t1 < n)
            def _():
                wait_lin(1)
                issue_gather(1)

            @pl.when(u > 0)
            def _():
                wait_out(0)

            compute(0)
            issue_out(t0, 0)

            @pl.when(t2 < n)
            def _():
                issue_lin(t2, 0)
                wait_lin(0)
                issue_gather(0)

            @pl.when(t1 < n)
            def _():
                wait_gather(1)

                @pl.when(u > 0)
                def _():
                    wait_out(1)

                compute(1)
                issue_out(t1, 1)

            @pl.when(t3 < n)
            def _():
                issue_lin(t3, 1)

            return carry

        lax.fori_loop(0, (n + 1) // 2, step, 0)
        wait_out(0)

        @pl.when(n > 1)
        def _():
            wait_out(1)

    def buf2(*shape_dtype):
        shape, dtype = shape_dtype
        return [pltpu.VMEM(shape, dtype), pltpu.VMEM(shape, dtype)]

    return pl.kernel(
        body,
        out_type=(jax.ShapeDtypeStruct((E,), jnp.float32),
                  jax.ShapeDtypeStruct((E,), jnp.float32),
                  jax.ShapeDtypeStruct((E,), jnp.float32),
                  jax.ShapeDtypeStruct((E,), jnp.float32)),
        mesh=mesh,
        scratch_types=[
            buf2((NSUB, SUB), jnp.int32),
            buf2((NSUB, SUB), jnp.int32),
            buf2((C,), jnp.float32),
            buf2((C,), jnp.float32),
            buf2((C,), jnp.float32),
            buf2((NSUB, SUB, TW), jnp.float32),
            buf2((NSUB, SUB, TW), jnp.float32),
            buf2((C,), jnp.float32),
            buf2((C,), jnp.float32),
            buf2((C,), jnp.float32),
            buf2((C,), jnp.float32),
            [pltpu.SemaphoreType.DMA, pltpu.SemaphoreType.DMA],
            [pltpu.SemaphoreType.DMA, pltpu.SemaphoreType.DMA],
            [pltpu.SemaphoreType.DMA, pltpu.SemaphoreType.DMA],
        ],
        compiler_params=pltpu.CompilerParams(needs_layout_passes=False,
                                             use_tc_tiling_on_sc=False),
        interpret=interpret,
    )


def kernel(pos, cell, cell_shift, batch, edge_index):
    N = pos.shape[0]
    E = edge_index.shape[1]
    cellf = cell.reshape(-1, 9)
    B = cellf.shape[0]
    C = 1024
    assert E % C == 0 and C % SUB == 0
    table = jnp.concatenate(
        [pos, cellf[batch], jnp.zeros((N, TW - 12), jnp.float32)], axis=1)
    vx, vy, vz, ln = _make(N, E, B, C)(
        table, cell_shift[:, 0], cell_shift[:, 1], cell_shift[:, 2],
        edge_index[0].reshape(-1, SUB), edge_index[1].reshape(-1, SUB))
    return jnp.stack([vx, vy, vz], axis=1), ln


# speedup vs baseline: 1.3596x; 1.3596x over previous
import functools

import jax
import jax.numpy as jnp
from jax import lax
from jax.experimental import pallas as pl
from jax.experimental.pallas import tpu as pltpu
from jax.experimental.pallas import tpu_sc as plsc

NC = 2
NS = 16
NW = NC * NS
L = 16
SUB = 128
TW = 16


@functools.lru_cache(maxsize=None)
def _make(N, E, B, C, interpret=False):
    del B
    G = C // L
    NSUB = C // SUB
    T = E // C

    mesh = plsc.VectorSubcoreMesh(core_axis_name="c", subcore_axis_name="s",
                                  num_cores=NC, num_subcores=NS)

    def body(table_hbm, cs0_hbm, cs1_hbm, cs2_hbm, src_hbm, dst_hbm,
             vx_hbm, vy_hbm, vz_hbm, len_hbm,
             sidx, didx, cs0, cs1, cs2, srow, drow, vxb, vyb, vzb, lnb,
             lsem, gsem, osem):
        wid = lax.axis_index("s") * NC + lax.axis_index("c")
        n = (T - wid + NW - 1) // NW

        def issue_lin(t, p):
            chunk = wid + t * NW
            base = chunk * C
            brow = chunk * NSUB
            pltpu.async_copy(src_hbm.at[pl.ds(brow, NSUB), :], sidx[p], lsem[p])
            pltpu.async_copy(dst_hbm.at[pl.ds(brow, NSUB), :], didx[p], lsem[p])
            pltpu.async_copy(cs0_hbm.at[pl.ds(base, C)], cs0[p], lsem[p])
            pltpu.async_copy(cs1_hbm.at[pl.ds(base, C)], cs1[p], lsem[p])
            pltpu.async_copy(cs2_hbm.at[pl.ds(base, C)], cs2[p], lsem[p])

        def wait_lin(p):
            pltpu.make_async_copy(src_hbm.at[pl.ds(0, NSUB), :], sidx[p], lsem[p]).wait()
            pltpu.make_async_copy(dst_hbm.at[pl.ds(0, NSUB), :], didx[p], lsem[p]).wait()
            pltpu.make_async_copy(cs0_hbm.at[pl.ds(0, C)], cs0[p], lsem[p]).wait()
            pltpu.make_async_copy(cs1_hbm.at[pl.ds(0, C)], cs1[p], lsem[p]).wait()
            pltpu.make_async_copy(cs2_hbm.at[pl.ds(0, C)], cs2[p], lsem[p]).wait()

        def issue_gather(p):
            for j in range(NSUB):
                pltpu.async_copy(table_hbm.at[sidx[p].at[j]], srow[p].at[j], gsem[p])
                pltpu.async_copy(table_hbm.at[didx[p].at[j]], drow[p].at[j], gsem[p])

        def wait_gather(p):
            for j in range(NSUB):
                pltpu.make_async_copy(table_hbm.at[sidx[p].at[j]], srow[p].at[j], gsem[p]).wait()
                pltpu.make_async_copy(table_hbm.at[didx[p].at[j]], drow[p].at[j], gsem[p]).wait()

        def issue_out(t, p):
            base = (wid + t * NW) * C
            pltpu.async_copy(vxb[p], vx_hbm.at[pl.ds(base, C)], osem[p])
            pltpu.async_copy(vyb[p], vy_hbm.at[pl.ds(base, C)], osem[p])
            pltpu.async_copy(vzb[p], vz_hbm.at[pl.ds(base, C)], osem[p])
            pltpu.async_copy(lnb[p], len_hbm.at[pl.ds(base, C)], osem[p])

        def wait_out(p):
            pltpu.make_async_copy(vxb[p], vx_hbm.at[pl.ds(0, C)], osem[p]).wait()
            pltpu.make_async_copy(vyb[p], vy_hbm.at[pl.ds(0, C)], osem[p]).wait()
            pltpu.make_async_copy(vzb[p], vz_hbm.at[pl.ds(0, C)], osem[p]).wait()
            pltpu.make_async_copy(lnb[p], len_hbm.at[pl.ds(0, C)], osem[p]).wait()

        def compute(p):
            def group(g, carry2):
                sl = pl.ds(g * L, L)
                rows = g * L + lax.iota(jnp.int32, L)
                jv = rows >> 7
                rv = rows & (SUB - 1)

                def scol(c):
                    return plsc.load_gather(
                        srow[p], [jv, rv, jnp.full((L,), c, jnp.int32)])

                def dcol(c):
                    return plsc.load_gather(
                        drow[p], [jv, rv, jnp.full((L,), c, jnp.int32)])

                dx = dcol(0) - scol(0)
                dy = dcol(1) - scol(1)
                dz = dcol(2) - scol(2)
                c0 = cs0[p][sl]
                c1 = cs1[p][sl]
                c2 = cs2[p][sl]
                vx = dx + c0 * scol(3) + c1 * scol(6) + c2 * scol(9)
                vy = dy + c0 * scol(4) + c1 * scol(7) + c2 * scol(10)
                vz = dz + c0 * scol(5) + c1 * scol(8) + c2 * scol(11)
                s = vx * vx + vy * vy + vz * vz
                i = plsc.bitcast(s, jnp.int32)
                y = plsc.bitcast(jnp.int32(0x5F3759DF) - (i >> 1), jnp.float32)
                for _ in range(3):
                    y = y * (1.5 - 0.5 * s * y * y)
                vxb[p][sl] = vx
                vyb[p][sl] = vy
                vzb[p][sl] = vz
                lnb[p][sl] = s * y
                return carry2

            lax.fori_loop(0, G, group, 0, unroll=2)

        issue_lin(0, 0)

        @pl.when(n > 1)
        def _():
            issue_lin(1, 1)

        wait_lin(0)
        issue_gather(0)

        def step(u, carry):
            t0 = 2 * u
            t1 = t0 + 1
            t2 = t0 + 2
            t3 = t0 + 3

            wait_gather(0)

            @pl.when(t1 < n)
            def _():
                wait_lin(1)
                issue_gather(1)

            @pl.when(u > 0)
            def _():
                wait_out(0)

            compute(0)
            issue_out(t0, 0)

            @pl.when(t2 < n)
            def _():
                issue_lin(t2, 0)
                wait_lin(0)
                issue_gather(0)

            @pl.when(t1 < n)
            def _():
                wait_gather(1)

                @pl.when(u > 0)
                def _():
                    wait_out(1)

                compute(1)
                issue_out(t1, 1)

            @pl.when(t3 < n)
            def _():
                issue_lin(t3, 1)

            return carry

        lax.fori_loop(0, (n + 1) // 2, step, 0)
        wait_out(0)

        @pl.when(n > 1)
        def _():
            wait_out(1)

    def buf2(*shape_dtype):
        shape, dtype = shape_dtype
        return [pltpu.VMEM(shape, dtype), pltpu.VMEM(shape, dtype)]

    return pl.kernel(
        body,
        out_type=(jax.ShapeDtypeStruct((E,), jnp.float32),
                  jax.ShapeDtypeStruct((E,), jnp.float32),
                  jax.ShapeDtypeStruct((E,), jnp.float32),
                  jax.ShapeDtypeStruct((E,), jnp.float32)),
        mesh=mesh,
        scratch_types=[
            buf2((NSUB, SUB), jnp.int32),
            buf2((NSUB, SUB), jnp.int32),
            buf2((C,), jnp.float32),
            buf2((C,), jnp.float32),
            buf2((C,), jnp.float32),
            buf2((NSUB, SUB, TW), jnp.float32),
            buf2((NSUB, SUB, TW), jnp.float32),
            buf2((C,), jnp.float32),
            buf2((C,), jnp.float32),
            buf2((C,), jnp.float32),
            buf2((C,), jnp.float32),
            [pltpu.SemaphoreType.DMA, pltpu.SemaphoreType.DMA],
            [pltpu.SemaphoreType.DMA, pltpu.SemaphoreType.DMA],
            [pltpu.SemaphoreType.DMA, pltpu.SemaphoreType.DMA],
        ],
        compiler_params=pltpu.CompilerParams(needs_layout_passes=False,
                                             use_tc_tiling_on_sc=False),
        interpret=interpret,
    )


def kernel(pos, cell, cell_shift, batch, edge_index):
    N = pos.shape[0]
    E = edge_index.shape[1]
    cellf = cell.reshape(-1, 9)
    B = cellf.shape[0]
    C = 1024
    assert E % C == 0 and C % SUB == 0
    onehot = (batch[:, None] == jnp.arange(B)[None, :]).astype(jnp.float32)
    cellmat = onehot @ cellf
    table = jnp.concatenate(
        [pos, cellmat, jnp.zeros((N, TW - 12), jnp.float32)], axis=1)
    vx, vy, vz, ln = _make(N, E, B, C)(
        table, cell_shift[:, 0], cell_shift[:, 1], cell_shift[:, 2],
        edge_index[0].reshape(-1, SUB), edge_index[1].reshape(-1, SUB))
    return jnp.stack([vx, vy, vz], axis=1), ln


# speedup vs baseline: 1.3855x; 1.0190x over previous
import functools

import jax
import jax.numpy as jnp
from jax import lax
from jax.experimental import pallas as pl
from jax.experimental.pallas import tpu as pltpu
from jax.experimental.pallas import tpu_sc as plsc

NC = 2
NS = 16
NW = NC * NS
L = 16
SUB = 128
TW = 16


@functools.lru_cache(maxsize=None)
def _make(N, E, B, C, interpret=False):
    del B
    G = C // L
    NSUB = C // SUB
    T = E // C

    mesh = plsc.VectorSubcoreMesh(core_axis_name="c", subcore_axis_name="s",
                                  num_cores=NC, num_subcores=NS)

    def body(table_hbm, cs0_hbm, cs1_hbm, cs2_hbm, ei_hbm,
             vx_hbm, vy_hbm, vz_hbm, len_hbm,
             sidx, didx, cs0, cs1, cs2, srow, drow, vxb, vyb, vzb, lnb,
             lsem, gsem, osem):
        wid = lax.axis_index("s") * NC + lax.axis_index("c")
        n = (T - wid + NW - 1) // NW

        def issue_lin(t, p):
            chunk = wid + t * NW
            base = chunk * C
            brow = chunk * NSUB
            pltpu.async_copy(ei_hbm.at[0, pl.ds(brow, NSUB), :], sidx[p], lsem[p])
            pltpu.async_copy(ei_hbm.at[1, pl.ds(brow, NSUB), :], didx[p], lsem[p])
            pltpu.async_copy(cs0_hbm.at[pl.ds(base, C)], cs0[p], lsem[p])
            pltpu.async_copy(cs1_hbm.at[pl.ds(base, C)], cs1[p], lsem[p])
            pltpu.async_copy(cs2_hbm.at[pl.ds(base, C)], cs2[p], lsem[p])

        def wait_lin(p):
            pltpu.make_async_copy(ei_hbm.at[0, pl.ds(0, NSUB), :], sidx[p], lsem[p]).wait()
            pltpu.make_async_copy(ei_hbm.at[1, pl.ds(0, NSUB), :], didx[p], lsem[p]).wait()
            pltpu.make_async_copy(cs0_hbm.at[pl.ds(0, C)], cs0[p], lsem[p]).wait()
            pltpu.make_async_copy(cs1_hbm.at[pl.ds(0, C)], cs1[p], lsem[p]).wait()
            pltpu.make_async_copy(cs2_hbm.at[pl.ds(0, C)], cs2[p], lsem[p]).wait()

        def issue_gather(p):
            for j in range(NSUB):
                pltpu.async_copy(table_hbm.at[sidx[p].at[j]], srow[p].at[j], gsem[p])
                pltpu.async_copy(table_hbm.at[didx[p].at[j]], drow[p].at[j], gsem[p])

        def wait_gather(p):
            for j in range(NSUB):
                pltpu.make_async_copy(table_hbm.at[sidx[p].at[j]], srow[p].at[j], gsem[p]).wait()
                pltpu.make_async_copy(table_hbm.at[didx[p].at[j]], drow[p].at[j], gsem[p]).wait()

        def issue_out(t, p):
            base = (wid + t * NW) * C
            pltpu.async_copy(vxb[p], vx_hbm.at[pl.ds(base, C)], osem[p])
            pltpu.async_copy(vyb[p], vy_hbm.at[pl.ds(base, C)], osem[p])
            pltpu.async_copy(vzb[p], vz_hbm.at[pl.ds(base, C)], osem[p])
            pltpu.async_copy(lnb[p], len_hbm.at[pl.ds(base, C)], osem[p])

        def wait_out(p):
            pltpu.make_async_copy(vxb[p], vx_hbm.at[pl.ds(0, C)], osem[p]).wait()
            pltpu.make_async_copy(vyb[p], vy_hbm.at[pl.ds(0, C)], osem[p]).wait()
            pltpu.make_async_copy(vzb[p], vz_hbm.at[pl.ds(0, C)], osem[p]).wait()
            pltpu.make_async_copy(lnb[p], len_hbm.at[pl.ds(0, C)], osem[p]).wait()

        def compute(p):
            def group(g, carry2):
                sl = pl.ds(g * L, L)
                rows = g * L + lax.iota(jnp.int32, L)
                jv = rows >> 7
                rv = rows & (SUB - 1)

                def scol(c):
                    return plsc.load_gather(
                        srow[p], [jv, rv, jnp.full((L,), c, jnp.int32)])

                def dcol(c):
                    return plsc.load_gather(
                        drow[p], [jv, rv, jnp.full((L,), c, jnp.int32)])

                dx = dcol(0) - scol(0)
                dy = dcol(1) - scol(1)
                dz = dcol(2) - scol(2)
                c0 = cs0[p][sl]
                c1 = cs1[p][sl]
                c2 = cs2[p][sl]
                vx = dx + c0 * scol(3) + c1 * scol(6) + c2 * scol(9)
                vy = dy + c0 * scol(4) + c1 * scol(7) + c2 * scol(10)
                vz = dz + c0 * scol(5) + c1 * scol(8) + c2 * scol(11)
                s = vx * vx + vy * vy + vz * vz
                i = plsc.bitcast(s, jnp.int32)
                y = plsc.bitcast(jnp.int32(0x5F3759DF) - (i >> 1), jnp.float32)
                for _ in range(3):
                    y = y * (1.5 - 0.5 * s * y * y)
                vxb[p][sl] = vx
                vyb[p][sl] = vy
                vzb[p][sl] = vz
                lnb[p][sl] = s * y
                return carry2

            lax.fori_loop(0, G, group, 0, unroll=2)

        issue_lin(0, 0)

        @pl.when(n > 1)
        def _():
            issue_lin(1, 1)

        wait_lin(0)
        issue_gather(0)

        def step(u, carry):
            t0 = 2 * u
            t1 = t0 + 1
            t2 = t0 + 2
            t3 = t0 + 3

            wait_gather(0)

            @pl.when(t1 < n)
            def _():
                wait_lin(1)
                issue_gather(1)

            @pl.when(u > 0)
            def _():
                wait_out(0)

            compute(0)
            issue_out(t0, 0)

            @pl.when(t2 < n)
            def _():
                issue_lin(t2, 0)
                wait_lin(0)
                issue_gather(0)

            @pl.when(t1 < n)
            def _():
                wait_gather(1)

                @pl.when(u > 0)
                def _():
                    wait_out(1)

                compute(1)
                issue_out(t1, 1)

            @pl.when(t3 < n)
            def _():
                issue_lin(t3, 1)

            return carry

        lax.fori_loop(0, (n + 1) // 2, step, 0)
        wait_out(0)

        @pl.when(n > 1)
        def _():
            wait_out(1)

    def buf2(*shape_dtype):
        shape, dtype = shape_dtype
        return [pltpu.VMEM(shape, dtype), pltpu.VMEM(shape, dtype)]

    return pl.kernel(
        body,
        out_type=(jax.ShapeDtypeStruct((E,), jnp.float32),
                  jax.ShapeDtypeStruct((E,), jnp.float32),
                  jax.ShapeDtypeStruct((E,), jnp.float32),
                  jax.ShapeDtypeStruct((E,), jnp.float32)),
        mesh=mesh,
        scratch_types=[
            buf2((NSUB, SUB), jnp.int32),
            buf2((NSUB, SUB), jnp.int32),
            buf2((C,), jnp.float32),
            buf2((C,), jnp.float32),
            buf2((C,), jnp.float32),
            buf2((NSUB, SUB, TW), jnp.float32),
            buf2((NSUB, SUB, TW), jnp.float32),
            buf2((C,), jnp.float32),
            buf2((C,), jnp.float32),
            buf2((C,), jnp.float32),
            buf2((C,), jnp.float32),
            [pltpu.SemaphoreType.DMA, pltpu.SemaphoreType.DMA],
            [pltpu.SemaphoreType.DMA, pltpu.SemaphoreType.DMA],
            [pltpu.SemaphoreType.DMA, pltpu.SemaphoreType.DMA],
        ],
        compiler_params=pltpu.CompilerParams(needs_layout_passes=False,
                                             use_tc_tiling_on_sc=False),
        interpret=interpret,
    )


def kernel(pos, cell, cell_shift, batch, edge_index):
    N = pos.shape[0]
    E = edge_index.shape[1]
    cellf = cell.reshape(-1, 9)
    B = cellf.shape[0]
    C = 1024
    assert E % C == 0 and C % SUB == 0
    onehot = (batch[:, None] == jnp.arange(B)[None, :]).astype(jnp.float32)
    cellmat = onehot @ cellf
    table = jnp.concatenate(
        [pos, cellmat, jnp.zeros((N, TW - 12), jnp.float32)], axis=1)
    vx, vy, vz, ln = _make(N, E, B, C)(
        table, cell_shift[:, 0], cell_shift[:, 1], cell_shift[:, 2],
        edge_index.reshape(2, -1, SUB))
    return jnp.stack([vx, vy, vz], axis=1), ln
